# baseline (speedup 1.0000x reference)
import functools

import jax
import jax.numpy as jnp
from jax import lax
from jax.experimental import pallas as pl
from jax.experimental.pallas import tpu as pltpu
from jax.experimental.pallas import tpu_sc as plsc

VOCAB = 1000000
EMBED = 64
B = 4096

_info = plsc.get_sparse_core_info()
_NC, _NS = _info.num_cores, _info.num_subcores
_NW = _NC * _NS
_BPW = B // _NW
_CHUNK = 16

_mesh = plsc.VectorSubcoreMesh(core_axis_name="c", subcore_axis_name="s")


def _make_gather():
    @functools.partial(
        pl.kernel,
        mesh=_mesh,
        out_type=jax.ShapeDtypeStruct((B, EMBED, 128), jnp.float32),
        scratch_types=[
            pltpu.VMEM((_BPW,), jnp.int32),
            *[pltpu.VMEM((EMBED, 128), jnp.float32) for _ in range(4)],
            *[pltpu.SemaphoreType.DMA for _ in range(8)],
        ],
    )
    def gather_k(idx_hbm, tab, g_hbm, idx_va,
                 ta0, ta1, ta2, ta3,
                 ia0, ia1, ia2, ia3,
                 oa0, oa1, oa2, oa3):
        ta = [ta0, ta1, ta2, ta3]
        sin_a = [ia0, ia1, ia2, ia3]
        sout_a = [oa0, oa1, oa2, oa3]
        wid = lax.axis_index("s") * _NC + lax.axis_index("c")
        base = wid * _BPW
        pltpu.sync_copy(idx_hbm.at[pl.ds(base, _BPW)], idx_va)

        def chunk_body(c, carry):
            j0 = c * _CHUNK
            veca = idx_va[pl.ds(j0, _CHUNK)]
            for u in range(_CHUNK + 2):
                slot = u % 4
                if 2 <= u:
                    w = u - 2
                    ws = w % 4
                    jw = base + j0 + w
                    pltpu.make_async_copy(tab.at[:, pl.ds(0, 128)],
                                          ta[ws], sin_a[ws]).wait()
                    pltpu.async_copy(ta[ws], g_hbm.at[jw], sout_a[ws])
                if u < _CHUNK:
                    if u >= 4:
                        jp = base + j0 + u - 4
                        pltpu.make_async_copy(ta[slot], g_hbm.at[jp],
                                              sout_a[slot]).wait()
                    offa = pl.multiple_of(veca[u] // 128 * 128, 128)
                    pltpu.async_copy(tab.at[:, pl.ds(offa, 128)],
                                     ta[slot], sin_a[slot])
            for w in range(_CHUNK - 4, _CHUNK):
                slot = w % 4
                jt = base + j0 + w
                pltpu.make_async_copy(ta[slot], g_hbm.at[jt],
                                      sout_a[slot]).wait()
            return carry

        lax.fori_loop(0, _BPW // _CHUNK, chunk_body, 0)

    return gather_k


_gather = _make_gather()

_BE = 256
_BM = 512


def _ex_body(c_ref, t_ref, o_ref):
    c = c_ref[...]
    lanes = lax.broadcasted_iota(jnp.int32, (_BE, 1, 128), 2)
    oh = (lanes == c[:, :, None]).astype(jnp.float32)
    o_ref[...] = jnp.sum(t_ref[...] * oh, axis=2)


def _extract(tiles, cols):
    return pl.pallas_call(
        _ex_body,
        grid=(B // _BE,),
        in_specs=[
            pl.BlockSpec((_BE, 1), lambda i: (i, 0)),
            pl.BlockSpec((_BE, EMBED, 128), lambda i: (i, 0, 0)),
        ],
        out_specs=pl.BlockSpec((_BE, EMBED), lambda i: (i, 0)),
        out_shape=jax.ShapeDtypeStruct((B, EMBED), jnp.float32),
    )(cols, tiles)


def _mm_body(a_ref, b_ref, o_ref):
    a = a_ref[...].astype(jnp.bfloat16)
    b = b_ref[...].astype(jnp.bfloat16)
    o_ref[...] = lax.dot_general(
        a, b, (((1,), (1,)), ((), ())),
        preferred_element_type=jnp.float32)


def kernel(target, context, in_embed, out_embed):
    tgt = target.astype(jnp.int32)
    ctx = context.astype(jnp.int32)
    ga = _gather(tgt, in_embed.T)
    gb = _gather(ctx, out_embed.T)
    in_embeds = _extract(ga, (tgt % 128).reshape(B, 1))
    out_embeds = _extract(gb, (ctx % 128).reshape(B, 1))
    scores = pl.pallas_call(
        _mm_body,
        grid=(B // _BM,),
        in_specs=[
            pl.BlockSpec((_BM, EMBED), lambda i: (i, 0)),
            pl.BlockSpec((B, EMBED), lambda i: (0, 0)),
        ],
        out_specs=pl.BlockSpec((_BM, B), lambda i: (i, 0)),
        out_shape=jax.ShapeDtypeStruct((B, B), jnp.float32),
    )(in_embeds, out_embeds)
    return scores


# speedup vs baseline: 1.0266x; 1.0266x over previous
import functools

import jax
import jax.numpy as jnp
from jax import lax
from jax.experimental import pallas as pl
from jax.experimental.pallas import tpu as pltpu
from jax.experimental.pallas import tpu_sc as plsc

VOCAB = 1000000
EMBED = 64
B = 4096

_info = plsc.get_sparse_core_info()
_NC, _NS = _info.num_cores, _info.num_subcores
_NW = _NC * _NS
_BPW = B // _NW
_CHUNK = 16

_mesh = plsc.VectorSubcoreMesh(core_axis_name="c", subcore_axis_name="s")


def _make_gather():
    @functools.partial(
        pl.kernel,
        mesh=_mesh,
        out_type=[
            jax.ShapeDtypeStruct((B, EMBED, 128), jnp.float32),
            jax.ShapeDtypeStruct((B, EMBED, 128), jnp.float32),
        ],
        scratch_types=[
            pltpu.VMEM((_BPW,), jnp.int32),
            pltpu.VMEM((_BPW,), jnp.int32),
            *[pltpu.VMEM((EMBED, 128), jnp.float32) for _ in range(8)],
            *[pltpu.SemaphoreType.DMA for _ in range(16)],
        ],
    )
    def gather_k(tgt_hbm, ctx_hbm, in_tab, out_tab, ga_hbm, gb_hbm,
                 idx_va, idx_vb,
                 ta0, ta1, ta2, ta3, tb0, tb1, tb2, tb3,
                 ia0, ia1, ia2, ia3, ib0, ib1, ib2, ib3,
                 oa0, oa1, oa2, oa3, ob0, ob1, ob2, ob3):
        ta = [ta0, ta1, ta2, ta3]
        tb = [tb0, tb1, tb2, tb3]
        sin_a = [ia0, ia1, ia2, ia3]
        sin_b = [ib0, ib1, ib2, ib3]
        sout_a = [oa0, oa1, oa2, oa3]
        sout_b = [ob0, ob1, ob2, ob3]
        wid = lax.axis_index("s") * _NC + lax.axis_index("c")
        base = wid * _BPW
        pltpu.sync_copy(tgt_hbm.at[pl.ds(base, _BPW)], idx_va)
        pltpu.sync_copy(ctx_hbm.at[pl.ds(base, _BPW)], idx_vb)

        def chunk_body(c, carry):
            j0 = c * _CHUNK
            veca = idx_va[pl.ds(j0, _CHUNK)]
            vecb = idx_vb[pl.ds(j0, _CHUNK)]
            for u in range(_CHUNK + 2):
                slot = u % 4
                if 2 <= u:
                    w = u - 2
                    ws = w % 4
                    jw = base + j0 + w
                    pltpu.make_async_copy(in_tab.at[:, pl.ds(0, 128)],
                                          ta[ws], sin_a[ws]).wait()
                    pltpu.make_async_copy(out_tab.at[:, pl.ds(0, 128)],
                                          tb[ws], sin_b[ws]).wait()
                    pltpu.async_copy(ta[ws], ga_hbm.at[jw], sout_a[ws])
                    pltpu.async_copy(tb[ws], gb_hbm.at[jw], sout_b[ws])
                if u < _CHUNK:
                    if u >= 4:
                        jp = base + j0 + u - 4
                        pltpu.make_async_copy(ta[slot], ga_hbm.at[jp],
                                              sout_a[slot]).wait()
                        pltpu.make_async_copy(tb[slot], gb_hbm.at[jp],
                                              sout_b[slot]).wait()
                    offa = pl.multiple_of(veca[u] // 128 * 128, 128)
                    offb = pl.multiple_of(vecb[u] // 128 * 128, 128)
                    pltpu.async_copy(in_tab.at[:, pl.ds(offa, 128)],
                                     ta[slot], sin_a[slot])
                    pltpu.async_copy(out_tab.at[:, pl.ds(offb, 128)],
                                     tb[slot], sin_b[slot])
            for w in range(_CHUNK - 4, _CHUNK):
                slot = w % 4
                jt = base + j0 + w
                pltpu.make_async_copy(ta[slot], ga_hbm.at[jt],
                                      sout_a[slot]).wait()
                pltpu.make_async_copy(tb[slot], gb_hbm.at[jt],
                                      sout_b[slot]).wait()
            return carry

        lax.fori_loop(0, _BPW // _CHUNK, chunk_body, 0)

    return gather_k


_gather = _make_gather()

_BE = 256
_BM = 512


def _ex_body(c_ref, t_ref, o_ref):
    c = c_ref[...]
    lanes = lax.broadcasted_iota(jnp.int32, (_BE, 1, 128), 2)
    oh = (lanes == c[:, :, None]).astype(jnp.float32)
    o_ref[...] = jnp.sum(t_ref[...] * oh, axis=2)


def _extract(tiles, cols):
    return pl.pallas_call(
        _ex_body,
        grid=(B // _BE,),
        in_specs=[
            pl.BlockSpec((_BE, 1), lambda i: (i, 0)),
            pl.BlockSpec((_BE, EMBED, 128), lambda i: (i, 0, 0)),
        ],
        out_specs=pl.BlockSpec((_BE, EMBED), lambda i: (i, 0)),
        out_shape=jax.ShapeDtypeStruct((B, EMBED), jnp.float32),
    )(cols, tiles)


def _mm_body(a_ref, b_ref, o_ref):
    a = a_ref[...].astype(jnp.bfloat16)
    b = b_ref[...].astype(jnp.bfloat16)
    o_ref[...] = lax.dot_general(
        a, b, (((1,), (1,)), ((), ())),
        preferred_element_type=jnp.float32)


def kernel(target, context, in_embed, out_embed):
    tgt = target.astype(jnp.int32)
    ctx = context.astype(jnp.int32)
    ga, gb = _gather(tgt, ctx, in_embed.T, out_embed.T)
    in_embeds = _extract(ga, (tgt % 128).reshape(B, 1))
    out_embeds = _extract(gb, (ctx % 128).reshape(B, 1))
    scores = pl.pallas_call(
        _mm_body,
        grid=(B // _BM,),
        in_specs=[
            pl.BlockSpec((_BM, EMBED), lambda i: (i, 0)),
            pl.BlockSpec((B, EMBED), lambda i: (0, 0)),
        ],
        out_specs=pl.BlockSpec((_BM, B), lambda i: (i, 0)),
        out_shape=jax.ShapeDtypeStruct((B, B), jnp.float32),
    )(in_embeds, out_embeds)
    return scores
